# Initial kernel scaffold; baseline (speedup 1.0000x reference)
import jax
import jax.numpy as jnp
from jax.experimental import pallas as pl


def kernel(x, lut_weight):
    raise NotImplementedError("write your pallas kernel here")


# speedup vs baseline: 1.8775x; 1.8775x over previous
import functools

import jax
import jax.numpy as jnp
from jax import lax
from jax.experimental import pallas as pl
from jax.experimental.pallas import tpu as pltpu
from jax.experimental.pallas import tpu_sc as plsc

CHUNK = 128
K = 5
NBUF = 2


@functools.lru_cache(maxsize=None)
def _make_kernel(B, D):
    info = plsc.get_sparse_core_info()
    NC, NS = info.num_cores, info.num_subcores
    NW = NC * NS
    b_per_w = B // NW
    n_chunks = b_per_w // CHUNK
    T = n_chunks // K
    assert B == NW * T * K * CHUNK and T >= 2 * NBUF and (T - NBUF) % NBUF == 0

    mesh = plsc.VectorSubcoreMesh(core_axis_name="c", subcore_axis_name="s")

    @functools.partial(
        pl.kernel,
        out_type=jax.ShapeDtypeStruct((B, D), jnp.float32),
        mesh=mesh,
        compiler_params=pltpu.CompilerParams(use_tc_tiling_on_sc=False),
        scratch_types=[
            pltpu.VMEM((n_chunks, CHUNK), jnp.int32),
            pltpu.VMEM((NBUF, K, CHUNK, D), jnp.float32),
            pltpu.SemaphoreType.DMA,
            pltpu.SemaphoreType.DMA,
            pltpu.SemaphoreType.DMA,
            pltpu.SemaphoreType.DMA,
        ],
    )
    def gather_kernel(x_hbm, table_hbm, out_hbm, idx_v, rows_v, g0, g1, s0, s1):
        gsem = (g0, g1)
        ssem = (s0, s1)
        wid = lax.axis_index("s") * NC + lax.axis_index("c")
        row0 = wid * b_per_w

        pltpu.sync_copy(x_hbm.at[wid], idx_v)

        def gathers(t, p):
            return [
                pltpu.make_async_copy(
                    table_hbm.at[idx_v.at[t * K + j]], rows_v.at[p, j], gsem[p]
                )
                for j in range(K)
            ]

        def stores(t, p):
            return [
                pltpu.make_async_copy(
                    rows_v.at[p, j],
                    out_hbm.at[pl.ds(row0 + (t * K + j) * CHUNK, CHUNK)],
                    ssem[p],
                )
                for j in range(K)
            ]

        for p in range(NBUF):
            for d in gathers(p, p):
                d.start()

        def step(t, p):
            for d in gathers(t, p):
                d.wait()
            for d in stores(t, p):
                d.start()
            for d in stores(t, p):
                d.wait()

        def body(i, _):
            t0 = NBUF * i
            for p in range(NBUF):
                step(t0 + p, p)
                for d in gathers(t0 + p + NBUF, p):
                    d.start()
            return _

        lax.fori_loop(0, (T - NBUF) // NBUF, body, None)
        for p in range(NBUF):
            step(T - NBUF + p, p)

    return gather_kernel


def kernel(x, lut_weight):
    B, H = x.shape
    D = lut_weight.shape[1]
    info = plsc.get_sparse_core_info()
    NW = info.num_cores * info.num_subcores
    n_chunks = (B * H) // (NW * CHUNK)
    idx = x.astype(jnp.int32).reshape(NW, n_chunks, CHUNK)
    out = _make_kernel(B * H, D)(idx, lut_weight)
    return out.reshape(B, H, D)
